# baseline (speedup 1.0000x reference)
import functools
import math

import jax
import jax.numpy as jnp
from jax.experimental import pallas as pl
from jax.experimental.pallas import tpu as pltpu

N = 10000
E = 320000
D = 128
NEG = jnp.float32(-jnp.inf)


def _mm_kernel(a_ref, b_ref, o_ref):
    o_ref[...] = jnp.dot(a_ref[...], b_ref[...],
                         preferred_element_type=jnp.float32)


def _mm(a, b, bm=512):
    m, k = a.shape
    _, n = b.shape
    return pl.pallas_call(
        _mm_kernel,
        grid=(pl.cdiv(m, bm),),
        in_specs=[
            pl.BlockSpec((bm, k), lambda i: (i, 0)),
            pl.BlockSpec((k, n), lambda i: (0, 0)),
        ],
        out_specs=pl.BlockSpec((bm, n), lambda i: (i, 0)),
        out_shape=jax.ShapeDtypeStruct((m, n), jnp.float32),
    )(a, b)


def _leaky(v, slope):
    return jnp.where(v >= 0, v, slope * v)


def _gat_masked(x, src, dst, alive, prm):
    h = _mm(x, prm['W'])
    al = h @ prm['a_src']
    be = h @ prm['a_dst']
    gmax = jnp.max(jnp.where(alive > 0, al, NEG))
    B = _leaky(gmax + be, 0.2)
    m_edge = alive[src] * alive[dst]
    e_edge = _leaky(al[src] + be[dst], 0.2)
    ee = jnp.where(m_edge > 0, jnp.exp(e_edge - B[dst]), 0.0)
    denom_e = jax.ops.segment_sum(ee, dst, num_segments=N)
    ee_self = jnp.exp(_leaky(al + be, 0.2) - B)
    denom = denom_e + ee_self
    num = jax.ops.segment_sum(ee[:, None] * h[src], dst, num_segments=N)
    num = num + ee_self[:, None] * h
    out = num / (denom + 1e-16)[:, None] + prm['b']
    return jnp.where(alive[:, None] > 0, jax.nn.relu(out), 0.0)


def _gc_masked(x, src, dst, alive, prm):
    m_edge = alive[src] * alive[dst]
    agg = jax.ops.segment_sum(x[src] * m_edge[:, None], dst, num_segments=N)
    out = _mm(agg, prm['W_rel']) + _mm(x, prm['W_root']) + prm['b']
    return jnp.where(alive[:, None] > 0, jax.nn.relu(out), 0.0)


def _pool_masked(x, alive, k, p):
    score = (x @ p) / (jnp.linalg.norm(p) + 1e-16)
    sc = jnp.where(alive > 0, score, NEG)
    vals, perm = jax.lax.top_k(sc, k)
    t = vals[k - 1]
    cnt_gt = jnp.sum((sc > t).astype(jnp.int32))
    is_t = sc == t
    rank_t = jnp.cumsum(is_t.astype(jnp.int32))
    sel = (sc > t) | (is_t & (rank_t <= k - cnt_gt))
    new_alive = sel.astype(jnp.float32)
    xk = jnp.where(sel[:, None], x * jnp.tanh(score)[:, None], 0.0)
    return xk, new_alive


def _readout_masked(x, alive, k):
    mx = jnp.max(jnp.where(alive[:, None] > 0, x, NEG), axis=0)
    mean = jnp.sum(x, axis=0) / jnp.float32(k)
    return jnp.concatenate([mx, mean])[None, :]


def kernel(x, edge_index, batch, params):
    src = edge_index[0]
    dst = edge_index[1]
    ones = jnp.ones((N,), jnp.float32)
    ks = [int(math.ceil(0.8 * N))]
    ks.append(int(math.ceil(0.8 * ks[0])))
    ks.append(int(math.ceil(0.8 * ks[1])))

    a = ones
    x0 = _gat_masked(x, src, dst, a, params['gat10'])
    x0, a = _pool_masked(x0, a, ks[0], params['pool20'])
    r1 = _readout_masked(x0, a, ks[0])
    x0 = _gat_masked(x0, src, dst, a, params['gat20'])
    x0, a = _pool_masked(x0, a, ks[1], params['pool20'])
    r2 = _readout_masked(x0, a, ks[1])
    x0 = _gat_masked(x0, src, dst, a, params['gat30'])
    x0, a = _pool_masked(x0, a, ks[2], params['pool30'])
    r3 = _readout_masked(x0, a, ks[2])

    c = ones
    z = _gc_masked(x, src, dst, c, params['gc11'])
    z, c = _pool_masked(z, c, ks[0], params['pool11'])
    z1 = _readout_masked(z, c, ks[0])
    z = _gc_masked(z, src, dst, c, params['gc21'])
    z, c = _pool_masked(z, c, ks[1], params['pool21'])
    z2 = _readout_masked(z, c, ks[1])
    z = _gc_masked(z, src, dst, c, params['gc31'])
    z, c = _pool_masked(z, c, ks[2], params['pool31'])
    z3 = _readout_masked(z, c, ks[2])

    h = r1 + r2 + r3 + z1 + z2 + z3
    h = jax.nn.relu(h @ params['lin1']['W'] + params['lin1']['b'])
    h = jax.nn.leaky_relu(h @ params['lin2']['W'] + params['lin2']['b'], 0.01)
    h = h @ params['lin3']['W'] + params['lin3']['b']
    return jax.nn.log_softmax(h, axis=-1)


# speedup vs baseline: 1.2363x; 1.2363x over previous
import functools
import math

import jax
import jax.numpy as jnp
from jax import lax
from jax.experimental import pallas as pl
from jax.experimental.pallas import tpu as pltpu
from jax.experimental.pallas import tpu_sc as plsc

N = 10000
E = 320000
D = 128
NEG = float('-inf')

NC, NS, NW = 2, 16, 32
CH = 80
CPT = 128
EP = CH * CPT * NW
N_ACC = 10240
RPT = N_ACC // NS
ZR = 32


def _sc_rows_segment_sum(h, src2d, dst2d):
    mesh = plsc.VectorSubcoreMesh(core_axis_name="c", subcore_axis_name="s")

    @functools.partial(
        pl.kernel,
        out_type=jax.ShapeDtypeStruct((NC, N_ACC, D), jnp.float32),
        mesh=mesh,
        scratch_types=[
            pltpu.VMEM((CPT, CH), jnp.int32),
            pltpu.VMEM((CPT, CH), jnp.int32),
            pltpu.VMEM((CH, D), jnp.float32),
            pltpu.VMEM((ZR, D), jnp.float32),
            pltpu.VMEM_SHARED((N_ACC, D), jnp.float32),
            pltpu.SemaphoreType.DMA,
        ],
    )
    def k(h_hbm, s_hbm, d_hbm, out_hbm, sbuf, dbuf, rows, zbuf, acc, sem):
        c = lax.axis_index("c")
        s = lax.axis_index("s")
        wid = c * NS + s
        zv = jnp.zeros((16,), jnp.float32)

        def zrow(i, _):
            for g in range(D // 16):
                zbuf[i, pl.ds(g * 16, 16)] = zv
            return 0

        lax.fori_loop(0, ZR, zrow, 0)
        for j in range(RPT // ZR):
            pltpu.sync_copy(zbuf, acc.at[pl.ds(s * RPT + j * ZR, ZR)])
        plsc.subcore_barrier()

        pltpu.sync_copy(s_hbm.at[pl.ds(wid * CPT, CPT)], sbuf)
        pltpu.sync_copy(d_hbm.at[pl.ds(wid * CPT, CPT)], dbuf)

        def chunk(j, _):
            pltpu.async_copy(h_hbm.at[sbuf.at[j]], rows, sem).wait()
            pltpu.sync_copy(rows, acc.at[dbuf.at[j]], add=True)
            return 0

        lax.fori_loop(0, CPT, chunk, 0)
        plsc.subcore_barrier()
        for j in range(RPT // ZR):
            base = s * RPT + j * ZR
            pltpu.sync_copy(acc.at[pl.ds(base, ZR)], zbuf)
            pltpu.sync_copy(zbuf, out_hbm.at[c].at[pl.ds(base, ZR)])

    return k(h, src2d, dst2d)


def _mm_kernel(a_ref, b_ref, o_ref):
    o_ref[...] = jnp.dot(a_ref[...], b_ref[...],
                         preferred_element_type=jnp.float32)


def _mm(a, b, bm=512):
    m, k = a.shape
    _, n = b.shape
    return pl.pallas_call(
        _mm_kernel,
        grid=(pl.cdiv(m, bm),),
        in_specs=[
            pl.BlockSpec((bm, k), lambda i: (i, 0)),
            pl.BlockSpec((k, n), lambda i: (0, 0)),
        ],
        out_specs=pl.BlockSpec((bm, n), lambda i: (i, 0)),
        out_shape=jax.ShapeDtypeStruct((m, n), jnp.float32),
    )(a, b)


def _leaky(v, slope):
    return jnp.where(v >= 0, v, slope * v)


def _gat_masked(x, src, dst, alive, prm):
    h = _mm(x, prm['W'])
    al = h @ prm['a_src']
    be = h @ prm['a_dst']
    gmax = jnp.max(jnp.where(alive > 0, al, NEG))
    B = _leaky(gmax + be, 0.2)
    m_edge = alive[src] * alive[dst]
    e_edge = _leaky(al[src] + be[dst], 0.2)
    ee = jnp.where(m_edge > 0, jnp.exp(e_edge - B[dst]), 0.0)
    denom_e = jax.ops.segment_sum(ee, dst, num_segments=N)
    ee_self = jnp.exp(_leaky(al + be, 0.2) - B)
    denom = denom_e + ee_self
    num = jax.ops.segment_sum(ee[:, None] * h[src], dst, num_segments=N)
    num = num + ee_self[:, None] * h
    out = num / (denom + 1e-16)[:, None] + prm['b']
    return jnp.where(alive[:, None] > 0, jax.nn.relu(out), 0.0)


def _gc_masked(x, src2d, dst2d, alive, prm):
    part = _sc_rows_segment_sum(x, src2d, dst2d)
    agg = part[0, :N] + part[1, :N]
    out = _mm(agg, prm['W_rel']) + _mm(x, prm['W_root']) + prm['b']
    return jnp.where(alive[:, None] > 0, jax.nn.relu(out), 0.0)


def _pool_masked(x, alive, k, p):
    score = (x @ p) / (jnp.linalg.norm(p) + 1e-16)
    sc = jnp.where(alive > 0, score, NEG)
    vals, perm = jax.lax.top_k(sc, k)
    t = vals[k - 1]
    cnt_gt = jnp.sum((sc > t).astype(jnp.int32))
    is_t = sc == t
    rank_t = jnp.cumsum(is_t.astype(jnp.int32))
    sel = (sc > t) | (is_t & (rank_t <= k - cnt_gt))
    new_alive = sel.astype(jnp.float32)
    xk = jnp.where(sel[:, None], x * jnp.tanh(score)[:, None], 0.0)
    return xk, new_alive


def _readout_masked(x, alive, k):
    mx = jnp.max(jnp.where(alive[:, None] > 0, x, NEG), axis=0)
    mean = jnp.sum(x, axis=0) / jnp.float32(k)
    return jnp.concatenate([mx, mean])[None, :]


def kernel(x, edge_index, batch, params):
    src = edge_index[0]
    dst = edge_index[1]
    ones = jnp.ones((N,), jnp.float32)
    ks = [int(math.ceil(0.8 * N))]
    ks.append(int(math.ceil(0.8 * ks[0])))
    ks.append(int(math.ceil(0.8 * ks[1])))

    a = ones
    x0 = _gat_masked(x, src, dst, a, params['gat10'])
    x0, a = _pool_masked(x0, a, ks[0], params['pool20'])
    r1 = _readout_masked(x0, a, ks[0])
    x0 = _gat_masked(x0, src, dst, a, params['gat20'])
    x0, a = _pool_masked(x0, a, ks[1], params['pool20'])
    r2 = _readout_masked(x0, a, ks[1])
    x0 = _gat_masked(x0, src, dst, a, params['gat30'])
    x0, a = _pool_masked(x0, a, ks[2], params['pool30'])
    r3 = _readout_masked(x0, a, ks[2])

    npad = EP - E
    src_p = jnp.concatenate([src, jnp.zeros((npad,), jnp.int32)])
    dst_p = jnp.concatenate([dst, N + (jnp.arange(npad, dtype=jnp.int32) % (N_ACC - N))])
    src2d = src_p.reshape(EP // CH, CH)
    dst2d = dst_p.reshape(EP // CH, CH)

    c = ones
    z = _gc_masked(x, src2d, dst2d, c, params['gc11'])
    z, c = _pool_masked(z, c, ks[0], params['pool11'])
    z1 = _readout_masked(z, c, ks[0])
    z = _gc_masked(z, src2d, dst2d, c, params['gc21'])
    z, c = _pool_masked(z, c, ks[1], params['pool21'])
    z2 = _readout_masked(z, c, ks[1])
    z = _gc_masked(z, src2d, dst2d, c, params['gc31'])
    z, c = _pool_masked(z, c, ks[2], params['pool31'])
    z3 = _readout_masked(z, c, ks[2])

    h = r1 + r2 + r3 + z1 + z2 + z3
    h = jax.nn.relu(h @ params['lin1']['W'] + params['lin1']['b'])
    h = jax.nn.leaky_relu(h @ params['lin2']['W'] + params['lin2']['b'], 0.01)
    h = h @ params['lin3']['W'] + params['lin3']['b']
    return jax.nn.log_softmax(h, axis=-1)


# speedup vs baseline: 12.0035x; 9.7096x over previous
import functools
import math

import jax
import jax.numpy as jnp
from jax import lax
from jax.experimental import pallas as pl
from jax.experimental.pallas import tpu as pltpu
from jax.experimental.pallas import tpu_sc as plsc

N = 10000
E = 320000
D = 128
NEG = float('-inf')

NC, NS, NW = 2, 16, 32
CH = 128
CPT = 80
EP = CH * CPT * NW
N_ACC = 10240
RPT = N_ACC // NS
ZR = 32


def _sc_rows_segment_sum(h, src2d, dst2d):
    mesh = plsc.VectorSubcoreMesh(core_axis_name="c", subcore_axis_name="s")

    @functools.partial(
        pl.kernel,
        out_type=jax.ShapeDtypeStruct((NC, N_ACC, D), jnp.float32),
        mesh=mesh,
        scratch_types=[
            pltpu.VMEM((CPT, CH), jnp.int32),
            pltpu.VMEM((CPT, CH), jnp.int32),
            pltpu.VMEM((CH, D), jnp.float32),
            pltpu.VMEM((ZR, D), jnp.float32),
            pltpu.VMEM_SHARED((N_ACC, D), jnp.float32),
            pltpu.SemaphoreType.DMA,
        ],
    )
    def k(h_hbm, s_hbm, d_hbm, out_hbm, sbuf, dbuf, rows, zbuf, acc, sem):
        c = lax.axis_index("c")
        s = lax.axis_index("s")
        wid = c * NS + s
        zv = jnp.zeros((16,), jnp.float32)

        def zrow(i, _):
            for g in range(D // 16):
                zbuf[i, pl.ds(g * 16, 16)] = zv
            return 0

        lax.fori_loop(0, ZR, zrow, 0)
        for j in range(RPT // ZR):
            pltpu.sync_copy(zbuf, acc.at[pl.ds(s * RPT + j * ZR, ZR)])
        plsc.subcore_barrier()

        pltpu.sync_copy(s_hbm.at[pl.ds(wid * CPT, CPT)], sbuf)
        pltpu.sync_copy(d_hbm.at[pl.ds(wid * CPT, CPT)], dbuf)

        def chunk(j, _):
            pltpu.async_copy(h_hbm.at[sbuf.at[j]], rows, sem).wait()
            pltpu.sync_copy(rows, acc.at[dbuf.at[j]], add=True)
            return 0

        lax.fori_loop(0, CPT, chunk, 0)
        plsc.subcore_barrier()
        for j in range(RPT // ZR):
            base = s * RPT + j * ZR
            pltpu.sync_copy(acc.at[pl.ds(base, ZR)], zbuf)
            pltpu.sync_copy(zbuf, out_hbm.at[c].at[pl.ds(base, ZR)])

    return k(h, src2d, dst2d)


def _sc_gat_edge(h, alp, bep, Bp, src2d, dst2d):
    mesh = plsc.VectorSubcoreMesh(core_axis_name="c", subcore_axis_name="s")
    G = 8

    @functools.partial(
        pl.kernel,
        out_type=(jax.ShapeDtypeStruct((NC, N_ACC, D), jnp.float32),
                  jax.ShapeDtypeStruct((NC, N_ACC), jnp.float32)),
        mesh=mesh,
        scratch_types=[
            pltpu.VMEM((G, CH), jnp.int32),
            pltpu.VMEM((G, CH), jnp.int32),
            pltpu.VMEM((G, CH), jnp.float32),
            pltpu.VMEM((G, CH), jnp.float32),
            pltpu.VMEM((G, CH), jnp.float32),
            pltpu.VMEM((CH,), jnp.float32),
            pltpu.VMEM((CH, D), jnp.float32),
            pltpu.VMEM((ZR, D), jnp.float32),
            pltpu.VMEM((RPT,), jnp.float32),
            pltpu.VMEM_SHARED((N_ACC, D), jnp.float32),
            pltpu.VMEM_SHARED((N_ACC,), jnp.float32),
            pltpu.SemaphoreType.DMA,
            pltpu.SemaphoreType.DMA,
        ],
    )
    def k(h_hbm, al_hbm, be_hbm, B_hbm, s_hbm, d_hbm, num_hbm, den_hbm,
          sidx, didx, alv, bev, Bv, eeb, rows, zbuf, zs, acc, dacc,
          semA, semR):
        c = lax.axis_index("c")
        s = lax.axis_index("s")
        wid = c * NS + s
        zv = jnp.zeros((16,), jnp.float32)

        def zrow(i, _):
            for g in range(D // 16):
                zbuf[i, pl.ds(g * 16, 16)] = zv
            return 0

        lax.fori_loop(0, ZR, zrow, 0)

        def zsc(i, _):
            zs[pl.ds(i * 16, 16)] = zv
            return 0

        lax.fori_loop(0, RPT // 16, zsc, 0)
        for j in range(RPT // ZR):
            pltpu.sync_copy(zbuf, acc.at[pl.ds(s * RPT + j * ZR, ZR)])
        pltpu.sync_copy(zs, dacc.at[pl.ds(s * RPT, RPT)])
        plsc.subcore_barrier()

        def group(gg, _):
            base = wid * CPT + gg * G
            pltpu.sync_copy(s_hbm.at[pl.ds(base, G)], sidx)
            pltpu.sync_copy(d_hbm.at[pl.ds(base, G)], didx)

            def chunk(b, _):
                pltpu.async_copy(al_hbm.at[sidx.at[b]], alv.at[b], semA).wait()
                pltpu.async_copy(be_hbm.at[didx.at[b]], bev.at[b], semA).wait()
                pltpu.async_copy(B_hbm.at[didx.at[b]], Bv.at[b], semA).wait()
                for t in range(CH // 16):
                    sl = pl.ds(t * 16, 16)
                    e = alv[b, sl] + bev[b, sl]
                    e = jnp.where(e >= 0, e, 0.2 * e)
                    eeb[sl] = jnp.exp(e - Bv[b, sl])
                pltpu.sync_copy(eeb, dacc.at[didx.at[b]], add=True)
                pltpu.async_copy(h_hbm.at[sidx.at[b]], rows, semR).wait()
                for t in range(CH // 16):
                    vv = eeb[pl.ds(t * 16, 16)]
                    for u in range(16):
                        i = t * 16 + u
                        ve = jnp.zeros((16,), jnp.float32) + vv[u]
                        for g in range(D // 16):
                            sl = pl.ds(g * 16, 16)
                            rows[i, sl] = rows[i, sl] * ve
                pltpu.sync_copy(rows, acc.at[didx.at[b]], add=True)
                return 0

            lax.fori_loop(0, G, chunk, 0)
            return 0

        lax.fori_loop(0, CPT // G, group, 0)
        plsc.subcore_barrier()
        for j in range(RPT // ZR):
            base = s * RPT + j * ZR
            pltpu.sync_copy(acc.at[pl.ds(base, ZR)], zbuf)
            pltpu.sync_copy(zbuf, num_hbm.at[c].at[pl.ds(base, ZR)])
        pltpu.sync_copy(dacc.at[pl.ds(s * RPT, RPT)], zs)
        pltpu.sync_copy(zs, den_hbm.at[c].at[pl.ds(s * RPT, RPT)])

    return k(h, alp, bep, Bp, src2d, dst2d)


def _mm_kernel(a_ref, b_ref, o_ref):
    o_ref[...] = jnp.dot(a_ref[...], b_ref[...],
                         preferred_element_type=jnp.float32)


def _mm(a, b, bm=512):
    m, k = a.shape
    _, n = b.shape
    return pl.pallas_call(
        _mm_kernel,
        grid=(pl.cdiv(m, bm),),
        in_specs=[
            pl.BlockSpec((bm, k), lambda i: (i, 0)),
            pl.BlockSpec((k, n), lambda i: (0, 0)),
        ],
        out_specs=pl.BlockSpec((bm, n), lambda i: (i, 0)),
        out_shape=jax.ShapeDtypeStruct((m, n), jnp.float32),
    )(a, b)


def _leaky(v, slope):
    return jnp.where(v >= 0, v, slope * v)


def _gat_masked(x, src2d, dst2d, alive, prm):
    h = _mm(x, prm['W'])
    al = h @ prm['a_src']
    be = h @ prm['a_dst']
    gmax = jnp.max(jnp.where(alive > 0, al, NEG))
    B = _leaky(gmax + be, 0.2)
    alp = jnp.where(alive > 0, al, -1e20)
    bep = jnp.where(alive > 0, be, -1e20)
    pad = jnp.zeros((N_ACC - N,), jnp.float32)
    nump, denp = _sc_gat_edge(
        h, jnp.concatenate([alp, pad]), jnp.concatenate([bep, pad]),
        jnp.concatenate([B, pad]), src2d, dst2d)
    ee_self = jnp.exp(_leaky(al + be, 0.2) - B)
    denom = denp[0, :N] + denp[1, :N] + ee_self
    num = nump[0, :N] + nump[1, :N] + ee_self[:, None] * h
    out = num / (denom + 1e-16)[:, None] + prm['b']
    return jnp.where(alive[:, None] > 0, jax.nn.relu(out), 0.0)


def _gc_masked(x, src2d, dst2d, alive, prm):
    part = _sc_rows_segment_sum(x, src2d, dst2d)
    agg = part[0, :N] + part[1, :N]
    out = _mm(agg, prm['W_rel']) + _mm(x, prm['W_root']) + prm['b']
    return jnp.where(alive[:, None] > 0, jax.nn.relu(out), 0.0)


def _pool_masked(x, alive, k, p):
    score = (x @ p) / (jnp.linalg.norm(p) + 1e-16)
    sc = jnp.where(alive > 0, score, NEG)
    vals, perm = jax.lax.top_k(sc, k)
    t = vals[k - 1]
    cnt_gt = jnp.sum((sc > t).astype(jnp.int32))
    is_t = sc == t
    rank_t = jnp.cumsum(is_t.astype(jnp.int32))
    sel = (sc > t) | (is_t & (rank_t <= k - cnt_gt))
    new_alive = sel.astype(jnp.float32)
    xk = jnp.where(sel[:, None], x * jnp.tanh(score)[:, None], 0.0)
    return xk, new_alive


def _readout_masked(x, alive, k):
    mx = jnp.max(jnp.where(alive[:, None] > 0, x, NEG), axis=0)
    mean = jnp.sum(x, axis=0) / jnp.float32(k)
    return jnp.concatenate([mx, mean])[None, :]


def kernel(x, edge_index, batch, params):
    src = edge_index[0]
    dst = edge_index[1]
    ones = jnp.ones((N,), jnp.float32)
    ks = [int(math.ceil(0.8 * N))]
    ks.append(int(math.ceil(0.8 * ks[0])))
    ks.append(int(math.ceil(0.8 * ks[1])))

    npad = EP - E
    src_p = jnp.concatenate([src, jnp.zeros((npad,), jnp.int32)])
    dst_p = jnp.concatenate([dst, N + (jnp.arange(npad, dtype=jnp.int32) % (N_ACC - N))])
    src2d = src_p.reshape(EP // CH, CH)
    dst2d = dst_p.reshape(EP // CH, CH)

    a = ones
    x0 = _gat_masked(x, src2d, dst2d, a, params['gat10'])
    x0, a = _pool_masked(x0, a, ks[0], params['pool20'])
    r1 = _readout_masked(x0, a, ks[0])
    x0 = _gat_masked(x0, src2d, dst2d, a, params['gat20'])
    x0, a = _pool_masked(x0, a, ks[1], params['pool20'])
    r2 = _readout_masked(x0, a, ks[1])
    x0 = _gat_masked(x0, src2d, dst2d, a, params['gat30'])
    x0, a = _pool_masked(x0, a, ks[2], params['pool30'])
    r3 = _readout_masked(x0, a, ks[2])

    c = ones
    z = _gc_masked(x, src2d, dst2d, c, params['gc11'])
    z, c = _pool_masked(z, c, ks[0], params['pool11'])
    z1 = _readout_masked(z, c, ks[0])
    z = _gc_masked(z, src2d, dst2d, c, params['gc21'])
    z, c = _pool_masked(z, c, ks[1], params['pool21'])
    z2 = _readout_masked(z, c, ks[1])
    z = _gc_masked(z, src2d, dst2d, c, params['gc31'])
    z, c = _pool_masked(z, c, ks[2], params['pool31'])
    z3 = _readout_masked(z, c, ks[2])

    h = r1 + r2 + r3 + z1 + z2 + z3
    h = jax.nn.relu(h @ params['lin1']['W'] + params['lin1']['b'])
    h = jax.nn.leaky_relu(h @ params['lin2']['W'] + params['lin2']['b'], 0.01)
    h = h @ params['lin3']['W'] + params['lin3']['b']
    return jax.nn.log_softmax(h, axis=-1)
